# Initial kernel scaffold; baseline (speedup 1.0000x reference)
import jax
import jax.numpy as jnp
from jax.experimental import pallas as pl


def kernel(x, emb):
    raise NotImplementedError("write your pallas kernel here")


# speedup vs baseline: 1.7184x; 1.7184x over previous
import jax
import jax.numpy as jnp
from jax.experimental import pallas as pl


_BS = 256


def _add_kernel(x_ref, e_ref, o_ref):
    o_ref[...] = x_ref[...] + e_ref[...][None, :, :]


def kernel(x, emb):
    B, S, D = x.shape
    grid = (S // _BS,)
    return pl.pallas_call(
        _add_kernel,
        grid=grid,
        in_specs=[
            pl.BlockSpec((B, _BS, D), lambda s: (0, s, 0)),
            pl.BlockSpec((_BS, D), lambda s: (s, 0)),
        ],
        out_specs=pl.BlockSpec((B, _BS, D), lambda s: (0, s, 0)),
        out_shape=jax.ShapeDtypeStruct(x.shape, x.dtype),
    )(x, emb[:S])


# speedup vs baseline: 1.7346x; 1.0094x over previous
import jax
import jax.numpy as jnp
from jax.experimental import pallas as pl


_BS = 512


def _add_kernel(x_ref, e_ref, o_ref):
    o_ref[...] = x_ref[...] + e_ref[...][None, :, :]


def kernel(x, emb):
    B, S, D = x.shape
    grid = (S // _BS,)
    return pl.pallas_call(
        _add_kernel,
        grid=grid,
        in_specs=[
            pl.BlockSpec((B, _BS, D), lambda s: (0, s, 0)),
            pl.BlockSpec((_BS, D), lambda s: (s, 0)),
        ],
        out_specs=pl.BlockSpec((B, _BS, D), lambda s: (0, s, 0)),
        out_shape=jax.ShapeDtypeStruct(x.shape, x.dtype),
    )(x, emb[:S])
